# baseline (speedup 1.0000x reference)
import functools

import jax
import jax.numpy as jnp
from jax import lax
from jax.experimental import pallas as pl
from jax.experimental.pallas import tpu as pltpu
from jax.experimental.pallas import tpu_sc as plsc

VOCAB = 1000000
DIM = 64
B = 4096
L = 200
N_CLASSES = 2

_INFO = plsc.get_sparse_core_info()
NC = _INFO.num_cores
NS = _INFO.num_subcores
NW = NC * NS
S_PER_W = B // NW
IDX_PER_W = S_PER_W * L
INV_L = 1.0 / L

_sc_mesh = plsc.VectorSubcoreMesh(core_axis_name="c", subcore_axis_name="s")


@functools.partial(
    pl.kernel,
    mesh=_sc_mesh,
    compiler_params=pltpu.CompilerParams(use_tc_tiling_on_sc=False),
    out_type=jax.ShapeDtypeStruct((B * DIM,), jnp.float32),
    scratch_types=[
        pltpu.VMEM((IDX_PER_W,), jnp.int32),
        pltpu.VMEM((L, DIM), jnp.float32),
        pltpu.VMEM((S_PER_W * DIM,), jnp.float32),
        pltpu.SemaphoreType.DMA,
    ],
)
def _sc_pool(x_hbm, table_hbm, out_hbm, idx_v, rows_v, pooled_v, sem):
    wid = lax.axis_index("s") * NC + lax.axis_index("c")
    pltpu.sync_copy(x_hbm.at[pl.ds(wid * IDX_PER_W, IDX_PER_W)], idx_v)

    def sentence_body(s, carry):
        cp = pltpu.async_copy(
            table_hbm.at[idx_v.at[pl.ds(s * L, L)]], rows_v, sem)
        cp.wait()

        zero = jnp.zeros((16,), jnp.float32)

        def tok_body(t, accs):
            a0, a1, a2, a3 = accs
            return (
                a0 + rows_v[t, pl.ds(0, 16)],
                a1 + rows_v[t, pl.ds(16, 16)],
                a2 + rows_v[t, pl.ds(32, 16)],
                a3 + rows_v[t, pl.ds(48, 16)],
            )

        a0, a1, a2, a3 = lax.fori_loop(0, L, tok_body, (zero,) * 4)
        base = s * DIM
        pooled_v[pl.ds(base, 16)] = a0 * INV_L
        pooled_v[pl.ds(base + 16, 16)] = a1 * INV_L
        pooled_v[pl.ds(base + 32, 16)] = a2 * INV_L
        pooled_v[pl.ds(base + 48, 16)] = a3 * INV_L
        return carry

    lax.fori_loop(0, S_PER_W, sentence_body, 0)
    pltpu.sync_copy(
        pooled_v, out_hbm.at[pl.ds(wid * S_PER_W * DIM, S_PER_W * DIM)])


def _linear_body(p_ref, wt_ref, b_ref, o_ref):
    o_ref[...] = (
        jnp.dot(p_ref[...], wt_ref[...], preferred_element_type=jnp.float32)
        + b_ref[...]
    )


_linear = pl.pallas_call(
    _linear_body,
    out_shape=jax.ShapeDtypeStruct((B, N_CLASSES), jnp.float32),
)


@jax.jit
def kernel(x, table, W, b):
    xf = x.reshape(-1).astype(jnp.int32)
    pooled = _sc_pool(xf, table).reshape(B, DIM)
    return _linear(pooled, W.T, b.reshape(1, N_CLASSES))


# speedup vs baseline: 1.1736x; 1.1736x over previous
import functools

import jax
import jax.numpy as jnp
from jax import lax
from jax.experimental import pallas as pl
from jax.experimental.pallas import tpu as pltpu
from jax.experimental.pallas import tpu_sc as plsc

VOCAB = 1000000
DIM = 64
B = 4096
L = 200
N_CLASSES = 2

_INFO = plsc.get_sparse_core_info()
NC = _INFO.num_cores
NS = _INFO.num_subcores
NW = NC * NS
S_PER_W = B // NW
IDX_PER_W = S_PER_W * L
INV_L = 1.0 / L

_sc_mesh = plsc.VectorSubcoreMesh(core_axis_name="c", subcore_axis_name="s")


@functools.partial(
    pl.kernel,
    mesh=_sc_mesh,
    compiler_params=pltpu.CompilerParams(use_tc_tiling_on_sc=False),
    out_type=jax.ShapeDtypeStruct((B * DIM,), jnp.float32),
    scratch_types=[
        pltpu.VMEM((IDX_PER_W,), jnp.int32),
        pltpu.VMEM((L, DIM), jnp.float32),
        pltpu.VMEM((L, DIM), jnp.float32),
        pltpu.VMEM((S_PER_W * DIM,), jnp.float32),
        pltpu.SemaphoreType.DMA,
        pltpu.SemaphoreType.DMA,
    ],
)
def _sc_pool(x_hbm, table_hbm, out_hbm, idx_v, rows0, rows1, pooled_v,
             sem0, sem1):
    wid = lax.axis_index("s") * NC + lax.axis_index("c")
    pltpu.sync_copy(x_hbm.at[pl.ds(wid * IDX_PER_W, IDX_PER_W)], idx_v)

    def start(s, rows, sem):
        pltpu.async_copy(table_hbm.at[idx_v.at[pl.ds(s * L, L)]], rows, sem)

    def wait(rows, sem):
        pltpu.make_async_copy(
            table_hbm.at[idx_v.at[pl.ds(0, L)]], rows, sem).wait()

    def accum_store(rows, s):
        zero = jnp.zeros((16,), jnp.float32)

        def tok8(i, accs):
            a0, a1, a2, a3, b0, b1, b2, b3 = accs
            t0 = i * 8
            for k in range(0, 8, 2):
                a0 = a0 + rows[t0 + k, pl.ds(0, 16)]
                a1 = a1 + rows[t0 + k, pl.ds(16, 16)]
                a2 = a2 + rows[t0 + k, pl.ds(32, 16)]
                a3 = a3 + rows[t0 + k, pl.ds(48, 16)]
                b0 = b0 + rows[t0 + k + 1, pl.ds(0, 16)]
                b1 = b1 + rows[t0 + k + 1, pl.ds(16, 16)]
                b2 = b2 + rows[t0 + k + 1, pl.ds(32, 16)]
                b3 = b3 + rows[t0 + k + 1, pl.ds(48, 16)]
            return (a0, a1, a2, a3, b0, b1, b2, b3)

        a0, a1, a2, a3, b0, b1, b2, b3 = lax.fori_loop(
            0, L // 8, tok8, (zero,) * 8)
        base = s * DIM
        pooled_v[pl.ds(base, 16)] = (a0 + b0) * INV_L
        pooled_v[pl.ds(base + 16, 16)] = (a1 + b1) * INV_L
        pooled_v[pl.ds(base + 32, 16)] = (a2 + b2) * INV_L
        pooled_v[pl.ds(base + 48, 16)] = (a3 + b3) * INV_L

    start(0, rows0, sem0)

    def pair_body(i, carry):
        s0 = 2 * i
        start(s0 + 1, rows1, sem1)
        wait(rows0, sem0)
        accum_store(rows0, s0)

        @pl.when(i < S_PER_W // 2 - 1)
        def _():
            start(s0 + 2, rows0, sem0)

        wait(rows1, sem1)
        accum_store(rows1, s0 + 1)
        return carry

    lax.fori_loop(0, S_PER_W // 2, pair_body, 0)
    pltpu.sync_copy(
        pooled_v, out_hbm.at[pl.ds(wid * S_PER_W * DIM, S_PER_W * DIM)])


def _linear_body(p_ref, wt_ref, b_ref, o_ref):
    o_ref[...] = (
        jnp.dot(p_ref[...], wt_ref[...], preferred_element_type=jnp.float32)
        + b_ref[...]
    )


_linear = pl.pallas_call(
    _linear_body,
    out_shape=jax.ShapeDtypeStruct((B, N_CLASSES), jnp.float32),
)


@jax.jit
def kernel(x, table, W, b):
    xf = x.reshape(-1).astype(jnp.int32)
    pooled = _sc_pool(xf, table).reshape(B, DIM)
    return _linear(pooled, W.T, b.reshape(1, N_CLASSES))


# speedup vs baseline: 5.1507x; 4.3888x over previous
import functools

import jax
import jax.numpy as jnp
from jax import lax
from jax.experimental import pallas as pl
from jax.experimental.pallas import tpu as pltpu
from jax.experimental.pallas import tpu_sc as plsc

VOCAB = 1000000
DIM = 64
B = 4096
L = 200
N_CLASSES = 2

_INFO = plsc.get_sparse_core_info()
NC = _INFO.num_cores
NS = _INFO.num_subcores
NW = NC * NS
S_PER_W = B // NW
INV_L = 1.0 / L

BLKN = 32768
NBLK = (VOCAB + BLKN - 1) // BLKN


def _proj_body(w_ref, t_ref, o0_ref, o1_ref):
    p = jnp.dot(w_ref[...], t_ref[...], preferred_element_type=jnp.float32)
    o0_ref[...] = p[0]
    o1_ref[...] = p[1]


_proj = pl.pallas_call(
    _proj_body,
    grid=(NBLK,),
    in_specs=[
        pl.BlockSpec((N_CLASSES, DIM), lambda i: (0, 0)),
        pl.BlockSpec((DIM, BLKN), lambda i: (0, i)),
    ],
    out_specs=[
        pl.BlockSpec((BLKN,), lambda i: (i,)),
        pl.BlockSpec((BLKN,), lambda i: (i,)),
    ],
    out_shape=[
        jax.ShapeDtypeStruct((VOCAB,), jnp.float32),
        jax.ShapeDtypeStruct((VOCAB,), jnp.float32),
    ],
)

_sc_mesh = plsc.VectorSubcoreMesh(core_axis_name="c", subcore_axis_name="s")


@functools.partial(
    pl.kernel,
    mesh=_sc_mesh,
    compiler_params=pltpu.CompilerParams(use_tc_tiling_on_sc=False),
    out_type=jax.ShapeDtypeStruct((N_CLASSES * B,), jnp.float32),
    scratch_types=[
        pltpu.VMEM((L, S_PER_W), jnp.int32),
        pltpu.VMEM((L * S_PER_W,), jnp.float32),
        pltpu.VMEM((L * S_PER_W,), jnp.float32),
        pltpu.VMEM((N_CLASSES * S_PER_W,), jnp.float32),
        pltpu.VMEM((16,), jnp.float32),
        pltpu.SemaphoreType.DMA,
        pltpu.SemaphoreType.DMA,
    ],
)
def _sc_pool(xt_hbm, p0_hbm, p1_hbm, b_hbm, out_hbm,
             idx_v, g0_v, g1_v, out_v, b_v, sem0, sem1):
    wid = lax.axis_index("s") * NC + lax.axis_index("c")
    base = wid * S_PER_W
    pltpu.sync_copy(xt_hbm.at[:, pl.ds(base, S_PER_W)], idx_v)
    pltpu.sync_copy(b_hbm, b_v)
    def fire(t, carry):
        pltpu.async_copy(
            p0_hbm.at[idx_v.at[t]], g0_v.at[pl.ds(t * S_PER_W, S_PER_W)],
            sem0)
        pltpu.async_copy(
            p1_hbm.at[idx_v.at[t]], g1_v.at[pl.ds(t * S_PER_W, S_PER_W)],
            sem1)
        return carry

    lax.fori_loop(0, L, fire, 0)
    pltpu.make_async_copy(p0_hbm.at[pl.ds(0, L * S_PER_W)], g0_v, sem0).wait()
    pltpu.make_async_copy(p1_hbm.at[pl.ds(0, L * S_PER_W)], g1_v, sem1).wait()

    zero = jnp.zeros((16,), jnp.float32)
    ngrp = S_PER_W // 16

    def tok_body(t, accs):
        new = []
        for j in range(ngrp):
            new.append(accs[j] + g0_v[pl.ds(t * S_PER_W + j * 16, 16)])
        for j in range(ngrp):
            new.append(
                accs[ngrp + j] + g1_v[pl.ds(t * S_PER_W + j * 16, 16)])
        return tuple(new)

    accs = lax.fori_loop(0, L, tok_body, (zero,) * (2 * ngrp))
    bvec = b_v[pl.ds(0, 16)]
    b0 = bvec[0]
    b1 = bvec[1]
    for j in range(ngrp):
        out_v[pl.ds(j * 16, 16)] = accs[j] * INV_L + b0
        out_v[pl.ds(S_PER_W + j * 16, 16)] = accs[ngrp + j] * INV_L + b1
    pltpu.sync_copy(out_v.at[pl.ds(0, S_PER_W)],
                    out_hbm.at[pl.ds(base, S_PER_W)])
    pltpu.sync_copy(out_v.at[pl.ds(S_PER_W, S_PER_W)],
                    out_hbm.at[pl.ds(B + base, S_PER_W)])


@jax.jit
def kernel(x, table, W, b):
    tableT = table.T
    p0, p1 = _proj(W, tableT)
    xt = x.T.astype(jnp.int32)
    bpad = jnp.zeros((16,), jnp.float32).at[:N_CLASSES].set(b)
    out_t = _sc_pool(xt, p0, p1, bpad)
    return out_t.reshape(N_CLASSES, B).T


# speedup vs baseline: 5.1816x; 1.0060x over previous
import functools

import jax
import jax.numpy as jnp
from jax import lax
from jax.experimental import pallas as pl
from jax.experimental.pallas import tpu as pltpu
from jax.experimental.pallas import tpu_sc as plsc

VOCAB = 1000000
DIM = 64
B = 4096
L = 200
N_CLASSES = 2

_INFO = plsc.get_sparse_core_info()
NC = _INFO.num_cores
NS = _INFO.num_subcores
NW = NC * NS
S_PER_W = B // NW
INV_L = 1.0 / L

BLKN = 65536
NBLK = (VOCAB + BLKN - 1) // BLKN


def _proj_body(w_ref, t_ref, o0_ref, o1_ref):
    p = jnp.dot(w_ref[...], t_ref[...], preferred_element_type=jnp.float32)
    o0_ref[...] = p[0]
    o1_ref[...] = p[1]


_proj = pl.pallas_call(
    _proj_body,
    grid=(NBLK,),
    in_specs=[
        pl.BlockSpec((N_CLASSES, DIM), lambda i: (0, 0)),
        pl.BlockSpec((DIM, BLKN), lambda i: (0, i)),
    ],
    out_specs=[
        pl.BlockSpec((BLKN,), lambda i: (i,)),
        pl.BlockSpec((BLKN,), lambda i: (i,)),
    ],
    out_shape=[
        jax.ShapeDtypeStruct((VOCAB,), jnp.float32),
        jax.ShapeDtypeStruct((VOCAB,), jnp.float32),
    ],
)

_sc_mesh = plsc.VectorSubcoreMesh(core_axis_name="c", subcore_axis_name="s")


@functools.partial(
    pl.kernel,
    mesh=_sc_mesh,
    compiler_params=pltpu.CompilerParams(use_tc_tiling_on_sc=False),
    out_type=jax.ShapeDtypeStruct((N_CLASSES * B,), jnp.float32),
    scratch_types=[
        pltpu.VMEM((L, S_PER_W), jnp.int32),
        pltpu.VMEM((L * S_PER_W,), jnp.float32),
        pltpu.VMEM((L * S_PER_W,), jnp.float32),
        pltpu.VMEM((N_CLASSES * S_PER_W,), jnp.float32),
        pltpu.VMEM((16,), jnp.float32),
        pltpu.SemaphoreType.DMA,
        pltpu.SemaphoreType.DMA,
    ],
)
def _sc_pool(xt_hbm, p0_hbm, p1_hbm, b_hbm, out_hbm,
             idx_v, g0_v, g1_v, out_v, b_v, sem0, sem1):
    wid = lax.axis_index("s") * NC + lax.axis_index("c")
    base = wid * S_PER_W
    pltpu.sync_copy(xt_hbm.at[:, pl.ds(base, S_PER_W)], idx_v)
    pltpu.sync_copy(b_hbm, b_v)
    NCHUNK = 8
    TCHUNK = L // NCHUNK
    CELEMS = TCHUNK * S_PER_W

    def fire(c):
        def fire_t(t, carry):
            pltpu.async_copy(
                p0_hbm.at[idx_v.at[t]], g0_v.at[pl.ds(t * S_PER_W, S_PER_W)],
                sem0)
            pltpu.async_copy(
                p1_hbm.at[idx_v.at[t]], g1_v.at[pl.ds(t * S_PER_W, S_PER_W)],
                sem1)
            return carry
        lax.fori_loop(c * TCHUNK, (c + 1) * TCHUNK, fire_t, 0)

    def drain(c):
        pltpu.make_async_copy(
            p0_hbm.at[pl.ds(0, CELEMS)],
            g0_v.at[pl.ds(c * CELEMS, CELEMS)], sem0).wait()
        pltpu.make_async_copy(
            p1_hbm.at[pl.ds(0, CELEMS)],
            g1_v.at[pl.ds(c * CELEMS, CELEMS)], sem1).wait()

    zero = jnp.zeros((16,), jnp.float32)
    ngrp = S_PER_W // 16

    def tok_body(t, accs):
        new = []
        for j in range(ngrp):
            new.append(accs[j] + g0_v[pl.ds(t * S_PER_W + j * 16, 16)])
        for j in range(ngrp):
            new.append(
                accs[ngrp + j] + g1_v[pl.ds(t * S_PER_W + j * 16, 16)])
        return tuple(new)

    fire(0)
    fire(1)
    accs = (zero,) * (2 * ngrp)
    for c in range(NCHUNK):
        if c + 2 < NCHUNK:
            fire(c + 2)
        drain(c)
        accs = lax.fori_loop(c * TCHUNK, (c + 1) * TCHUNK, tok_body, accs)
    bvec = b_v[pl.ds(0, 16)]
    b0 = bvec[0]
    b1 = bvec[1]
    for j in range(ngrp):
        out_v[pl.ds(j * 16, 16)] = accs[j] * INV_L + b0
        out_v[pl.ds(S_PER_W + j * 16, 16)] = accs[ngrp + j] * INV_L + b1
    pltpu.sync_copy(out_v.at[pl.ds(0, S_PER_W)],
                    out_hbm.at[pl.ds(base, S_PER_W)])
    pltpu.sync_copy(out_v.at[pl.ds(S_PER_W, S_PER_W)],
                    out_hbm.at[pl.ds(B + base, S_PER_W)])


@jax.jit
def kernel(x, table, W, b):
    tableT = table.T
    p0, p1 = _proj(W, tableT)
    xt = x.T.astype(jnp.int32)
    bpad = jnp.zeros((16,), jnp.float32).at[:N_CLASSES].set(b)
    out_t = _sc_pool(xt, p0, p1, bpad)
    return out_t.reshape(N_CLASSES, B).T
